# baseline (speedup 1.0000x reference)
import functools

import jax
import jax.numpy as jnp
import numpy as np
from jax import lax
from jax.experimental import pallas as pl
from jax.experimental.pallas import tpu as pltpu
from jax.experimental.pallas import tpu_sc as plsc

N = 50000
E = 800000
DIM = 64
NCONV = 3
TYPE_NUM = 100
CLS_DIM = 2000
NGRAPHS = 128
CUTOFF = 5.0
WIDTH = 1.0
N_CENTERS = int(np.ceil((CUTOFF - 0.0) / WIDTH))
GAP = float(CUTOFF / (N_CENTERS - 1))

BN = 2000
BE = 8000

NQ = 4
QW = DIM // NQ
SC_TILES = 16
TE = E // SC_TILES
W = 400
NWIN = TE // W
CH = 80
NCH = W // CH
NPAD = 50048
RPT = NPAD // SC_TILES
ZROWS = 136

_F32 = jnp.float32
_HIGH = lax.Precision.HIGHEST


def _dot(a, b):
    return jnp.dot(a, b, precision=_HIGH, preferred_element_type=_F32)


def _sp(x):
    return jax.nn.softplus(x)


def _sp_half(x):
    return 2.0 * jax.nn.softplus(0.5 * x)


def _embed_nn0_body(nt_ref, emb_ref, w1_ref, node_ref, nn0_ref):
    ids = nt_ref[0, 0, :]
    onehot = (ids[:, None] == lax.broadcasted_iota(jnp.int32, (BN, TYPE_NUM), 1)
              ).astype(_F32)
    nodeb = _dot(onehot, emb_ref[...])
    node_ref[...] = nodeb
    nn = _dot(nodeb, w1_ref[...])
    for q in range(NQ):
        nn0_ref[q] = nn[:, q * QW:(q + 1) * QW]


def _embed_nn0(node_type3, emb, w1_0):
    return pl.pallas_call(
        _embed_nn0_body,
        grid=(N // BN,),
        in_specs=[
            pl.BlockSpec((1, 1, BN), lambda i: (i, 0, 0)),
            pl.BlockSpec((TYPE_NUM, DIM), lambda i: (0, 0)),
            pl.BlockSpec((DIM, DIM), lambda i: (0, 0)),
        ],
        out_specs=[
            pl.BlockSpec((BN, DIM), lambda i: (i, 0)),
            pl.BlockSpec((NQ, BN, QW), lambda i: (0, i, 0)),
        ],
        out_shape=[
            jax.ShapeDtypeStruct((N, DIM), _F32),
            jax.ShapeDtypeStruct((NQ, NPAD, QW), _F32),
        ],
    )(node_type3, emb, w1_0)


def _h_body(d_ref, w1_ref, b1_ref, w2_ref, b2_ref, h_ref):
    d = d_ref[0, 0, :][:, None]
    cent = lax.broadcasted_iota(jnp.int32, (1, N_CENTERS), 1).astype(_F32) * GAP
    rbf = jnp.exp((-1.0 / GAP) * (d - cent) ** 2)
    pre = _dot(rbf, w1_ref[...]) + b1_ref[0, :]
    hs = _sp_half(pre)
    h = _dot(hs, w2_ref[...]) + b2_ref[0, :]
    zpad = jnp.zeros((BE // 8, 8, 128 - DIM), _F32)
    h_ref[...] = jnp.concatenate(
        [jnp.reshape(h, (BE // 8, 8, DIM)), zpad], axis=2)


def _h_conv(dist3, cf_w1, cf_b1, cf_w2, cf_b2):
    return pl.pallas_call(
        _h_body,
        grid=(E // BE,),
        in_specs=[
            pl.BlockSpec((1, 1, BE), lambda i: (i, 0, 0)),
            pl.BlockSpec((N_CENTERS, DIM), lambda i: (0, 0)),
            pl.BlockSpec((1, DIM), lambda i: (0, 0)),
            pl.BlockSpec((DIM, DIM), lambda i: (0, 0)),
            pl.BlockSpec((1, DIM), lambda i: (0, 0)),
        ],
        out_specs=pl.BlockSpec((BE // 8, 8, 128), lambda i: (i, 0, 0)),
        out_shape=jax.ShapeDtypeStruct((E // 8, 8, 128), _F32),
    )(dist3, cf_w1, cf_b1, cf_w2, cf_b2)


def _edge_conv_sc(nn4, h4, sd4):
    mesh = plsc.VectorSubcoreMesh(core_axis_name="c", subcore_axis_name="s")

    @functools.partial(
        pl.kernel,
        out_type=jax.ShapeDtypeStruct((NQ * NPAD, QW), _F32),
        mesh=mesh,
        scratch_types=[
            pltpu.VMEM((NCH, CH), jnp.int32),
            pltpu.VMEM((NCH, CH), jnp.int32),
            pltpu.VMEM((W, QW), _F32),
            pltpu.VMEM((W, QW), _F32),
            pltpu.VMEM((W // 8, 8, QW), _F32),
            pltpu.VMEM((W // 8, 8, QW), _F32),
            pltpu.VMEM_SHARED((NPAD, QW), _F32),
            pltpu.VMEM_SHARED((NPAD, QW), _F32),
            pltpu.SemaphoreType.DMA,
            pltpu.SemaphoreType.DMA,
        ],
        compiler_params=pltpu.CompilerParams(use_tc_tiling_on_sc=False),
    )
    def k(nn_hbm, h_hbm, sd_hbm, out_hbm, srcv, dstv, gbuf_a, gbuf_b,
          hbuf_a, hbuf_b, stab, acc, sem_g, sem_h):
        c = lax.axis_index("c")
        s = lax.axis_index("s")

        def issue(q, w, gb, hb):
            pltpu.sync_copy(sd_hbm.at[0, s * NWIN + w, pl.ds(0, NCH)], srcv)
            pltpu.async_copy(
                h_hbm.at[pl.ds((s * TE + w * W) // 8, W // 8),
                         :, pl.ds(q * QW, QW)],
                hb, sem_h)
            for j in range(NCH):
                pltpu.async_copy(stab.at[srcv.at[j]],
                                 gb.at[pl.ds(j * CH, CH)], sem_g)

        def drain(gb, hb):
            for j in range(NCH):
                pltpu.make_async_copy(nn_hbm.at[pl.ds(0, CH)],
                                      gb.at[pl.ds(j * CH, CH)], sem_g).wait()
            pltpu.make_async_copy(
                h_hbm.at[pl.ds(0, W // 8), :, pl.ds(0, QW)], hb, sem_h).wait()

        def compute(q, w, gb, hb):
            @pl.loop(0, W // 8)
            def _mul(r):
                for u in range(8):
                    gb.at[r * 8 + u][...] = (
                        gb.at[r * 8 + u][...] * hb.at[r, u][...])

            pltpu.sync_copy(sd_hbm.at[1, s * NWIN + w, pl.ds(0, NCH)], dstv)
            for j in range(NCH):
                pltpu.sync_copy(gb.at[pl.ds(j * CH, CH)],
                                acc.at[dstv.at[j]], add=True)

        for p in range(NQ // 2):
            q = c * (NQ // 2) + p

            @pl.loop(0, RPT // ZROWS)
            def _stage(i):
                pltpu.sync_copy(
                    nn_hbm.at[pl.ds(q * NPAD + s * RPT + i * ZROWS, ZROWS)],
                    stab.at[pl.ds(s * RPT + i * ZROWS, ZROWS)])

            @pl.loop(0, ZROWS)
            def _zero_rows(i):
                gbuf_a.at[i][...] = jnp.zeros((QW,), _F32)

            @pl.loop(0, RPT // ZROWS)
            def _zero_acc(i):
                pltpu.sync_copy(gbuf_a.at[pl.ds(0, ZROWS)],
                                acc.at[pl.ds(s * RPT + i * ZROWS, ZROWS)])

            plsc.subcore_barrier()

            issue(q, 0, gbuf_a, hbuf_a)

            @pl.loop(0, (NWIN - 1) // 2)
            def _win(m):
                w = m * 2
                drain(gbuf_a, hbuf_a)
                issue(q, w + 1, gbuf_b, hbuf_b)
                compute(q, w, gbuf_a, hbuf_a)
                drain(gbuf_b, hbuf_b)
                issue(q, w + 2, gbuf_a, hbuf_a)
                compute(q, w + 1, gbuf_b, hbuf_b)

            drain(gbuf_a, hbuf_a)
            compute(q, NWIN - 1, gbuf_a, hbuf_a)

            plsc.subcore_barrier()

            @pl.loop(0, RPT // ZROWS)
            def _dump(i):
                pltpu.sync_copy(
                    acc.at[pl.ds(s * RPT + i * ZROWS, ZROWS)],
                    out_hbm.at[pl.ds(q * NPAD + s * RPT + i * ZROWS, ZROWS)])

    return k(nn4, h4, sd4)


def _update_body_next(agg_ref, node_ref, n2w_ref, n2b_ref, n3w_ref, n3b_ref,
                      w1n_ref, nodeo_ref, nno_ref):
    agg = jnp.concatenate([agg_ref[q] for q in range(NQ)], axis=1)
    cf1 = _dot(agg, n2w_ref[...]) + n2b_ref[0, :]
    nodep = node_ref[...] + _dot(_sp_half(cf1), n3w_ref[...]) + n3b_ref[0, :]
    nodeo_ref[...] = nodep
    nn = _dot(nodep, w1n_ref[...])
    for q in range(NQ):
        nno_ref[q] = nn[:, q * QW:(q + 1) * QW]


def _update_body_last(agg_ref, node_ref, n2w_ref, n2b_ref, n3w_ref, n3b_ref,
                      nodeo_ref):
    agg = jnp.concatenate([agg_ref[q] for q in range(NQ)], axis=1)
    cf1 = _dot(agg, n2w_ref[...]) + n2b_ref[0, :]
    nodeo_ref[...] = (node_ref[...] + _dot(_sp_half(cf1), n3w_ref[...])
                      + n3b_ref[0, :])


def _update(agg4, node, n2w, n2b, n3w, n3b, w1n):
    wspec = pl.BlockSpec((DIM, DIM), lambda i: (0, 0))
    bspec = pl.BlockSpec((1, DIM), lambda i: (0, 0))
    in_specs = [
        pl.BlockSpec((NQ, BN, QW), lambda i: (0, i, 0)),
        pl.BlockSpec((BN, DIM), lambda i: (i, 0)),
        wspec, bspec, wspec, bspec,
    ]
    if w1n is None:
        return pl.pallas_call(
            _update_body_last,
            grid=(N // BN,),
            in_specs=in_specs,
            out_specs=pl.BlockSpec((BN, DIM), lambda i: (i, 0)),
            out_shape=jax.ShapeDtypeStruct((N, DIM), _F32),
        )(agg4, node, n2w, n2b, n3w, n3b)
    return pl.pallas_call(
        _update_body_next,
        grid=(N // BN,),
        in_specs=in_specs + [wspec],
        out_specs=[
            pl.BlockSpec((BN, DIM), lambda i: (i, 0)),
            pl.BlockSpec((NQ, BN, QW), lambda i: (0, i, 0)),
        ],
        out_shape=[
            jax.ShapeDtypeStruct((N, DIM), _F32),
            jax.ShapeDtypeStruct((NQ, NPAD, QW), _F32),
        ],
    )(agg4, node, n2w, n2b, n3w, n3b, w1n)


def _heads_body(node_ref, gid_ref, d1w_ref, d1b_ref, d2w_ref, d2b_ref,
                acw_ref, acb_ref, ap_ref, gsum_ref, cnt_ref):
    b = pl.program_id(0)
    atom = _sp(_dot(node_ref[...], d1w_ref[...]) + d1b_ref[0, :]) - np.log(2.0)
    res = _dot(atom, d2w_ref[...]) + d2b_ref[0, :]
    ap_ref[...] = _dot(jnp.maximum(res, 0.0), acw_ref[...]) + acb_ref[0, :]
    gids = gid_ref[0, 0, :]
    onehot = (gids[:, None] == lax.broadcasted_iota(jnp.int32, (BN, NGRAPHS), 1)
              ).astype(_F32)
    part = lax.dot_general(onehot, res, (((0,), (0,)), ((), ())),
                           precision=_HIGH, preferred_element_type=_F32)
    pcnt = jnp.sum(onehot, axis=0)[None, :]

    @pl.when(b == 0)
    def _init():
        gsum_ref[...] = jnp.zeros_like(gsum_ref)
        cnt_ref[...] = jnp.zeros_like(cnt_ref)

    gsum_ref[...] += part
    cnt_ref[...] += pcnt


def _heads(node, gid3, d1w, d1b, d2w, d2b, acw, acb):
    return pl.pallas_call(
        _heads_body,
        grid=(N // BN,),
        in_specs=[
            pl.BlockSpec((BN, DIM), lambda i: (i, 0)),
            pl.BlockSpec((1, 1, BN), lambda i: (i, 0, 0)),
            pl.BlockSpec((DIM, 256), lambda i: (0, 0)),
            pl.BlockSpec((1, 256), lambda i: (0, 0)),
            pl.BlockSpec((256, 256), lambda i: (0, 0)),
            pl.BlockSpec((1, 256), lambda i: (0, 0)),
            pl.BlockSpec((256, TYPE_NUM), lambda i: (0, 0)),
            pl.BlockSpec((1, TYPE_NUM), lambda i: (0, 0)),
        ],
        out_specs=[
            pl.BlockSpec((BN, TYPE_NUM), lambda i: (i, 0)),
            pl.BlockSpec((NGRAPHS, 256), lambda i: (0, 0)),
            pl.BlockSpec((1, NGRAPHS), lambda i: (0, 0)),
        ],
        out_shape=[
            jax.ShapeDtypeStruct((N, TYPE_NUM), _F32),
            jax.ShapeDtypeStruct((NGRAPHS, 256), _F32),
            jax.ShapeDtypeStruct((1, NGRAPHS), _F32),
        ],
    )(node, gid3, d1w, d1b, d2w, d2b, acw, acb)


def _cls_body(gsum_ref, cnt_ref, clsw_ref, clsb_ref, out_ref):
    counts = jnp.maximum(cnt_ref[0, :], 1.0)
    mean = gsum_ref[...] * (1.0 / counts)[:, None]
    out_ref[...] = _dot(mean, clsw_ref[...]) + clsb_ref[0, :]


def _cls(gsum, cnt, clsw, clsb):
    return pl.pallas_call(
        _cls_body,
        grid=(1,),
        in_specs=[
            pl.BlockSpec((NGRAPHS, 256), lambda i: (0, 0)),
            pl.BlockSpec((1, NGRAPHS), lambda i: (0, 0)),
            pl.BlockSpec((256, CLS_DIM), lambda i: (0, 0)),
            pl.BlockSpec((1, CLS_DIM), lambda i: (0, 0)),
        ],
        out_specs=pl.BlockSpec((NGRAPHS, CLS_DIM), lambda i: (0, 0)),
        out_shape=jax.ShapeDtypeStruct((NGRAPHS, CLS_DIM), _F32),
    )(gsum, cnt, clsw, clsb)


def kernel(node_type, edge_index, distance, graph_ids, emb, conv_w1, cf_w1,
           cf_b1, cf_w2, cf_b2, n2_w, n2_b, n3_w, n3_b, d1_w, d1_b, d2_w,
           d2_b, ac_w, ac_b, cls_w, cls_b):
    node_type3 = node_type.astype(jnp.int32).reshape(N // BN, 1, BN)
    gid3 = graph_ids.astype(jnp.int32).reshape(N // BN, 1, BN)
    dist3 = distance.astype(_F32).reshape(E // BE, 1, BE)
    ei = edge_index.astype(jnp.int32)
    sd4 = jnp.concatenate(
        [ei.reshape(2, E // W, NCH, CH),
         jnp.zeros((2, E // W, 8 - NCH, CH), jnp.int32)], axis=2)

    b1 = cf_b1.reshape(NCONV, 1, DIM)
    b2 = cf_b2.reshape(NCONV, 1, DIM)
    n2b = n2_b.reshape(NCONV, 1, DIM)
    n3b = n3_b.reshape(NCONV, 1, DIM)

    node, nn = _embed_nn0(node_type3, emb, conv_w1[0])
    hs = [_h_conv(dist3, cf_w1[i], b1[i], cf_w2[i], b2[i])
          for i in range(NCONV)]
    for i in range(NCONV):
        agg = _edge_conv_sc(nn.reshape(NQ * NPAD, QW),
                            hs[i], sd4)
        agg4 = agg.reshape(NQ, NPAD, QW)
        w1n = conv_w1[i + 1] if i + 1 < NCONV else None
        if w1n is None:
            node = _update(agg4, node, n2_w[i], n2b[i], n3_w[i], n3b[i], None)
        else:
            node, nn = _update(agg4, node, n2_w[i], n2b[i], n3_w[i], n3b[i],
                               w1n)

    atoms_preds, gsum, cnt = _heads(node, gid3, d1_w, d1_b.reshape(1, 256),
                                    d2_w, d2_b.reshape(1, 256), ac_w,
                                    ac_b.reshape(1, TYPE_NUM))
    cls_preds = _cls(gsum, cnt, cls_w, cls_b.reshape(1, CLS_DIM))
    return (atoms_preds, cls_preds)


# speedup vs baseline: 1.2233x; 1.2233x over previous
import functools

import jax
import jax.numpy as jnp
import numpy as np
from jax import lax
from jax.experimental import pallas as pl
from jax.experimental.pallas import tpu as pltpu
from jax.experimental.pallas import tpu_sc as plsc

N = 50000
E = 800000
DIM = 64
NCONV = 3
TYPE_NUM = 100
CLS_DIM = 2000
NGRAPHS = 128
CUTOFF = 5.0
WIDTH = 1.0
N_CENTERS = int(np.ceil((CUTOFF - 0.0) / WIDTH))
GAP = float(CUTOFF / (N_CENTERS - 1))

BN = 2000
BE = 8000

NQ = 4
QW = DIM // NQ
SC_TILES = 16
TE = E // SC_TILES
W = 400
NWIN = TE // W
CH = 80
NCH = W // CH
NPAD = 50048
RPT = NPAD // SC_TILES
ZROWS = 136

_F32 = jnp.float32
_HIGH = lax.Precision.HIGHEST


def _dot(a, b):
    return jnp.dot(a, b, precision=_HIGH, preferred_element_type=_F32)


def _sp(x):
    return jax.nn.softplus(x)


def _sp_half(x):
    return 2.0 * jax.nn.softplus(0.5 * x)


def _embed_nn0_body(nt_ref, emb_ref, w1_ref, node_ref, nn0_ref):
    ids = nt_ref[0, 0, :]
    onehot = (ids[:, None] == lax.broadcasted_iota(jnp.int32, (BN, TYPE_NUM), 1)
              ).astype(_F32)
    nodeb = _dot(onehot, emb_ref[...])
    node_ref[...] = nodeb
    nn = _dot(nodeb, w1_ref[...])
    for q in range(NQ):
        nn0_ref[q] = nn[:, q * QW:(q + 1) * QW]


def _embed_nn0(node_type3, emb, w1_0):
    return pl.pallas_call(
        _embed_nn0_body,
        grid=(N // BN,),
        in_specs=[
            pl.BlockSpec((1, 1, BN), lambda i: (i, 0, 0)),
            pl.BlockSpec((TYPE_NUM, DIM), lambda i: (0, 0)),
            pl.BlockSpec((DIM, DIM), lambda i: (0, 0)),
        ],
        out_specs=[
            pl.BlockSpec((BN, DIM), lambda i: (i, 0)),
            pl.BlockSpec((NQ, BN, QW), lambda i: (0, i, 0)),
        ],
        out_shape=[
            jax.ShapeDtypeStruct((N, DIM), _F32),
            jax.ShapeDtypeStruct((NQ, NPAD, QW), _F32),
        ],
    )(node_type3, emb, w1_0)


def _h_body(d_ref, w1_ref, b1_ref, w2_ref, b2_ref, h_ref):
    d = d_ref[0, 0, :][:, None]
    cent = lax.broadcasted_iota(jnp.int32, (1, N_CENTERS), 1).astype(_F32) * GAP
    rbf = jnp.exp((-1.0 / GAP) * (d - cent) ** 2)
    pre = b1_ref[0, :] + jnp.zeros((BE, DIM), _F32)
    for k in range(N_CENTERS):
        pre = pre + rbf[:, k:k + 1] * w1_ref[k:k + 1, :]
    hs = _sp_half(pre)
    h = _dot(hs, w2_ref[...]) + b2_ref[0, :]
    zpad = jnp.zeros((BE // 8, 8, 128 - DIM), _F32)
    h_ref[...] = jnp.concatenate(
        [jnp.reshape(h, (BE // 8, 8, DIM)), zpad], axis=2)


def _h_conv(dist3, cf_w1, cf_b1, cf_w2, cf_b2):
    return pl.pallas_call(
        _h_body,
        grid=(E // BE,),
        in_specs=[
            pl.BlockSpec((1, 1, BE), lambda i: (i, 0, 0)),
            pl.BlockSpec((N_CENTERS, DIM), lambda i: (0, 0)),
            pl.BlockSpec((1, DIM), lambda i: (0, 0)),
            pl.BlockSpec((DIM, DIM), lambda i: (0, 0)),
            pl.BlockSpec((1, DIM), lambda i: (0, 0)),
        ],
        out_specs=pl.BlockSpec((BE // 8, 8, 128), lambda i: (i, 0, 0)),
        out_shape=jax.ShapeDtypeStruct((E // 8, 8, 128), _F32),
    )(dist3, cf_w1, cf_b1, cf_w2, cf_b2)


def _edge_conv_sc(nn4, h4, sd4):
    mesh = plsc.VectorSubcoreMesh(core_axis_name="c", subcore_axis_name="s")

    @functools.partial(
        pl.kernel,
        out_type=jax.ShapeDtypeStruct((NQ * NPAD, QW), _F32),
        mesh=mesh,
        scratch_types=[
            pltpu.VMEM((NCH, CH), jnp.int32),
            pltpu.VMEM((NCH, CH), jnp.int32),
            pltpu.VMEM((W, QW), _F32),
            pltpu.VMEM((W, QW), _F32),
            pltpu.VMEM((W // 8, 8, QW), _F32),
            pltpu.VMEM((W // 8, 8, QW), _F32),
            pltpu.VMEM_SHARED((NPAD, QW), _F32),
            pltpu.VMEM_SHARED((NPAD, QW), _F32),
            pltpu.SemaphoreType.DMA,
            pltpu.SemaphoreType.DMA,
        ],
        compiler_params=pltpu.CompilerParams(use_tc_tiling_on_sc=False),
    )
    def k(nn_hbm, h_hbm, sd_hbm, out_hbm, srcv, dstv, gbuf_a, gbuf_b,
          hbuf_a, hbuf_b, stab, acc, sem_g, sem_h):
        c = lax.axis_index("c")
        s = lax.axis_index("s")

        def issue(q, w, gb, hb):
            pltpu.sync_copy(sd_hbm.at[0, s * NWIN + w, pl.ds(0, NCH)], srcv)
            pltpu.async_copy(
                h_hbm.at[pl.ds((s * TE + w * W) // 8, W // 8),
                         :, pl.ds(q * QW, QW)],
                hb, sem_h)
            for j in range(NCH):
                pltpu.async_copy(stab.at[srcv.at[j]],
                                 gb.at[pl.ds(j * CH, CH)], sem_g)

        def drain(gb, hb):
            for j in range(NCH):
                pltpu.make_async_copy(nn_hbm.at[pl.ds(0, CH)],
                                      gb.at[pl.ds(j * CH, CH)], sem_g).wait()
            pltpu.make_async_copy(
                h_hbm.at[pl.ds(0, W // 8), :, pl.ds(0, QW)], hb, sem_h).wait()

        def compute(q, w, gb, hb):
            @pl.loop(0, W // 8)
            def _mul(r):
                for u in range(8):
                    gb.at[r * 8 + u][...] = (
                        gb.at[r * 8 + u][...] * hb.at[r, u][...])

            pltpu.sync_copy(sd_hbm.at[1, s * NWIN + w, pl.ds(0, NCH)], dstv)
            for j in range(NCH):
                pltpu.sync_copy(gb.at[pl.ds(j * CH, CH)],
                                acc.at[dstv.at[j]], add=True)

        for p in range(NQ // 2):
            q = c * (NQ // 2) + p

            @pl.loop(0, RPT // ZROWS)
            def _stage(i):
                pltpu.sync_copy(
                    nn_hbm.at[pl.ds(q * NPAD + s * RPT + i * ZROWS, ZROWS)],
                    stab.at[pl.ds(s * RPT + i * ZROWS, ZROWS)])

            @pl.loop(0, ZROWS)
            def _zero_rows(i):
                gbuf_a.at[i][...] = jnp.zeros((QW,), _F32)

            @pl.loop(0, RPT // ZROWS)
            def _zero_acc(i):
                pltpu.sync_copy(gbuf_a.at[pl.ds(0, ZROWS)],
                                acc.at[pl.ds(s * RPT + i * ZROWS, ZROWS)])

            plsc.subcore_barrier()

            issue(q, 0, gbuf_a, hbuf_a)

            @pl.loop(0, (NWIN - 1) // 2)
            def _win(m):
                w = m * 2
                drain(gbuf_a, hbuf_a)
                issue(q, w + 1, gbuf_b, hbuf_b)
                compute(q, w, gbuf_a, hbuf_a)
                drain(gbuf_b, hbuf_b)
                issue(q, w + 2, gbuf_a, hbuf_a)
                compute(q, w + 1, gbuf_b, hbuf_b)

            drain(gbuf_a, hbuf_a)
            compute(q, NWIN - 1, gbuf_a, hbuf_a)

            plsc.subcore_barrier()

            @pl.loop(0, RPT // ZROWS)
            def _dump(i):
                pltpu.sync_copy(
                    acc.at[pl.ds(s * RPT + i * ZROWS, ZROWS)],
                    out_hbm.at[pl.ds(q * NPAD + s * RPT + i * ZROWS, ZROWS)])

    return k(nn4, h4, sd4)


def _update_body_next(agg_ref, node_ref, n2w_ref, n2b_ref, n3w_ref, n3b_ref,
                      w1n_ref, nodeo_ref, nno_ref):
    agg = jnp.concatenate([agg_ref[q] for q in range(NQ)], axis=1)
    cf1 = _dot(agg, n2w_ref[...]) + n2b_ref[0, :]
    nodep = node_ref[...] + _dot(_sp_half(cf1), n3w_ref[...]) + n3b_ref[0, :]
    nodeo_ref[...] = nodep
    nn = _dot(nodep, w1n_ref[...])
    for q in range(NQ):
        nno_ref[q] = nn[:, q * QW:(q + 1) * QW]


def _update_body_last(agg_ref, node_ref, n2w_ref, n2b_ref, n3w_ref, n3b_ref,
                      nodeo_ref):
    agg = jnp.concatenate([agg_ref[q] for q in range(NQ)], axis=1)
    cf1 = _dot(agg, n2w_ref[...]) + n2b_ref[0, :]
    nodeo_ref[...] = (node_ref[...] + _dot(_sp_half(cf1), n3w_ref[...])
                      + n3b_ref[0, :])


def _update(agg4, node, n2w, n2b, n3w, n3b, w1n):
    wspec = pl.BlockSpec((DIM, DIM), lambda i: (0, 0))
    bspec = pl.BlockSpec((1, DIM), lambda i: (0, 0))
    in_specs = [
        pl.BlockSpec((NQ, BN, QW), lambda i: (0, i, 0)),
        pl.BlockSpec((BN, DIM), lambda i: (i, 0)),
        wspec, bspec, wspec, bspec,
    ]
    if w1n is None:
        return pl.pallas_call(
            _update_body_last,
            grid=(N // BN,),
            in_specs=in_specs,
            out_specs=pl.BlockSpec((BN, DIM), lambda i: (i, 0)),
            out_shape=jax.ShapeDtypeStruct((N, DIM), _F32),
        )(agg4, node, n2w, n2b, n3w, n3b)
    return pl.pallas_call(
        _update_body_next,
        grid=(N // BN,),
        in_specs=in_specs + [wspec],
        out_specs=[
            pl.BlockSpec((BN, DIM), lambda i: (i, 0)),
            pl.BlockSpec((NQ, BN, QW), lambda i: (0, i, 0)),
        ],
        out_shape=[
            jax.ShapeDtypeStruct((N, DIM), _F32),
            jax.ShapeDtypeStruct((NQ, NPAD, QW), _F32),
        ],
    )(agg4, node, n2w, n2b, n3w, n3b, w1n)


def _heads_body(node_ref, gid_ref, d1w_ref, d1b_ref, d2w_ref, d2b_ref,
                acw_ref, acb_ref, ap_ref, gsum_ref, cnt_ref):
    b = pl.program_id(0)
    atom = _sp(_dot(node_ref[...], d1w_ref[...]) + d1b_ref[0, :]) - np.log(2.0)
    res = _dot(atom, d2w_ref[...]) + d2b_ref[0, :]
    ap_ref[...] = _dot(jnp.maximum(res, 0.0), acw_ref[...]) + acb_ref[0, :]
    gids = gid_ref[0, 0, :]
    onehot = (gids[:, None] == lax.broadcasted_iota(jnp.int32, (BN, NGRAPHS), 1)
              ).astype(_F32)
    part = lax.dot_general(onehot, res, (((0,), (0,)), ((), ())),
                           precision=_HIGH, preferred_element_type=_F32)
    pcnt = jnp.sum(onehot, axis=0)[None, :]

    @pl.when(b == 0)
    def _init():
        gsum_ref[...] = jnp.zeros_like(gsum_ref)
        cnt_ref[...] = jnp.zeros_like(cnt_ref)

    gsum_ref[...] += part
    cnt_ref[...] += pcnt


def _heads(node, gid3, d1w, d1b, d2w, d2b, acw, acb):
    return pl.pallas_call(
        _heads_body,
        grid=(N // BN,),
        in_specs=[
            pl.BlockSpec((BN, DIM), lambda i: (i, 0)),
            pl.BlockSpec((1, 1, BN), lambda i: (i, 0, 0)),
            pl.BlockSpec((DIM, 256), lambda i: (0, 0)),
            pl.BlockSpec((1, 256), lambda i: (0, 0)),
            pl.BlockSpec((256, 256), lambda i: (0, 0)),
            pl.BlockSpec((1, 256), lambda i: (0, 0)),
            pl.BlockSpec((256, TYPE_NUM), lambda i: (0, 0)),
            pl.BlockSpec((1, TYPE_NUM), lambda i: (0, 0)),
        ],
        out_specs=[
            pl.BlockSpec((BN, TYPE_NUM), lambda i: (i, 0)),
            pl.BlockSpec((NGRAPHS, 256), lambda i: (0, 0)),
            pl.BlockSpec((1, NGRAPHS), lambda i: (0, 0)),
        ],
        out_shape=[
            jax.ShapeDtypeStruct((N, TYPE_NUM), _F32),
            jax.ShapeDtypeStruct((NGRAPHS, 256), _F32),
            jax.ShapeDtypeStruct((1, NGRAPHS), _F32),
        ],
    )(node, gid3, d1w, d1b, d2w, d2b, acw, acb)


def _cls_body(gsum_ref, cnt_ref, clsw_ref, clsb_ref, out_ref):
    counts = jnp.maximum(cnt_ref[0, :], 1.0)
    mean = gsum_ref[...] * (1.0 / counts)[:, None]
    out_ref[...] = _dot(mean, clsw_ref[...]) + clsb_ref[0, :]


def _cls(gsum, cnt, clsw, clsb):
    return pl.pallas_call(
        _cls_body,
        grid=(1,),
        in_specs=[
            pl.BlockSpec((NGRAPHS, 256), lambda i: (0, 0)),
            pl.BlockSpec((1, NGRAPHS), lambda i: (0, 0)),
            pl.BlockSpec((256, CLS_DIM), lambda i: (0, 0)),
            pl.BlockSpec((1, CLS_DIM), lambda i: (0, 0)),
        ],
        out_specs=pl.BlockSpec((NGRAPHS, CLS_DIM), lambda i: (0, 0)),
        out_shape=jax.ShapeDtypeStruct((NGRAPHS, CLS_DIM), _F32),
    )(gsum, cnt, clsw, clsb)


def kernel(node_type, edge_index, distance, graph_ids, emb, conv_w1, cf_w1,
           cf_b1, cf_w2, cf_b2, n2_w, n2_b, n3_w, n3_b, d1_w, d1_b, d2_w,
           d2_b, ac_w, ac_b, cls_w, cls_b):
    node_type3 = node_type.astype(jnp.int32).reshape(N // BN, 1, BN)
    gid3 = graph_ids.astype(jnp.int32).reshape(N // BN, 1, BN)
    dist3 = distance.astype(_F32).reshape(E // BE, 1, BE)
    ei = edge_index.astype(jnp.int32)
    sd4 = jnp.concatenate(
        [ei.reshape(2, E // W, NCH, CH),
         jnp.zeros((2, E // W, 8 - NCH, CH), jnp.int32)], axis=2)

    b1 = cf_b1.reshape(NCONV, 1, DIM)
    b2 = cf_b2.reshape(NCONV, 1, DIM)
    n2b = n2_b.reshape(NCONV, 1, DIM)
    n3b = n3_b.reshape(NCONV, 1, DIM)

    node, nn = _embed_nn0(node_type3, emb, conv_w1[0])
    hs = [_h_conv(dist3, cf_w1[i], b1[i], cf_w2[i], b2[i])
          for i in range(NCONV)]
    for i in range(NCONV):
        agg = _edge_conv_sc(nn.reshape(NQ * NPAD, QW),
                            hs[i], sd4)
        agg4 = agg.reshape(NQ, NPAD, QW)
        w1n = conv_w1[i + 1] if i + 1 < NCONV else None
        if w1n is None:
            node = _update(agg4, node, n2_w[i], n2b[i], n3_w[i], n3b[i], None)
        else:
            node, nn = _update(agg4, node, n2_w[i], n2b[i], n3_w[i], n3b[i],
                               w1n)

    atoms_preds, gsum, cnt = _heads(node, gid3, d1_w, d1_b.reshape(1, 256),
                                    d2_w, d2_b.reshape(1, 256), ac_w,
                                    ac_b.reshape(1, TYPE_NUM))
    cls_preds = _cls(gsum, cnt, cls_w, cls_b.reshape(1, CLS_DIM))
    return (atoms_preds, cls_preds)
